# Initial kernel scaffold; baseline (speedup 1.0000x reference)
import jax
import jax.numpy as jnp
from jax.experimental import pallas as pl


def kernel(x, cb0, cb1, cb2, cb3):
    raise NotImplementedError("write your pallas kernel here")


# speedup vs baseline: 2.1939x; 2.1939x over previous
import jax
import jax.numpy as jnp
import numpy as np
from jax.experimental import pallas as pl

_TH = 8
_P = _TH * 112
_K = 1024
_C = 192
_HB = 112 // _TH


def _split_dot(a, b):
    a0 = a.astype(jnp.bfloat16).astype(jnp.float32)
    r1 = a - a0
    a1 = r1.astype(jnp.bfloat16).astype(jnp.float32)
    a2 = r1 - a1
    dn = (((1,), (0,)), ((), ()))
    out = jax.lax.dot_general(a0, b, dn, preferred_element_type=jnp.float32)
    for part in (a1, a2):
        out = out + jax.lax.dot_general(part, b, dn,
                                        preferred_element_type=jnp.float32)
    return out


def _vrq_body(x_ref, cbs_ref, gp_ref, q_ref, idx_ref, cnt_ref, sq_ref):
    b = pl.program_id(0)
    hb = pl.program_id(1)

    @pl.when(jnp.logical_and(b == 0, hb == 0))
    def _init():
        cnt_ref[...] = jnp.zeros_like(cnt_ref)
        sq_ref[...] = jnp.zeros_like(sq_ref)

    t = x_ref[0]
    top = t[:, :, :224]
    bot = t[:, :, 224:]
    hs = (top + bot).reshape(_C * _TH, 224)
    hd = (top - bot).reshape(_C * _TH, 224)
    gp = gp_ref[0]
    gm = gp_ref[1]
    subbands = (
        _split_dot(hs, gp),
        _split_dot(hd, gp),
        _split_dot(hs, gm),
        _split_dot(hd, gm),
    )

    iota_k = jax.lax.broadcasted_iota(jnp.int32, (_K, _P), 0)
    q_acc = None
    for s in range(4):
        z3 = subbands[s].reshape(_C, _TH, 112)
        z = jnp.concatenate([z3[:, th, :] for th in range(_TH)], axis=1)
        cb = cbs_ref[s]
        cn = jnp.sum(cb * cb, axis=1, keepdims=True)
        zn = jnp.sum(z * z, axis=0, keepdims=True)
        m = jax.lax.dot_general(cb, z, (((1,), (0,)), ((), ())),
                                preferred_element_type=jnp.float32)
        dist = (zn + cn) - 2.0 * m
        idx = jnp.argmin(dist, axis=0).astype(jnp.int32)
        onehot = (iota_k == idx[None, :]).astype(jnp.float32)
        cbh = cb.astype(jnp.bfloat16).astype(jnp.float32)
        cbl = cb - cbh
        dn0 = (((0,), (0,)), ((), ()))
        zq = (jax.lax.dot_general(cbh, onehot, dn0,
                                  preferred_element_type=jnp.float32)
              + jax.lax.dot_general(cbl, onehot, dn0,
                                    preferred_element_type=jnp.float32))
        r = zq - z
        sq_ref[s] = sq_ref[s] + jnp.sum((r * r).reshape(_C, _P // 128, 128),
                                        axis=(0, 1))
        cnt_ref[s] = cnt_ref[s] + jnp.sum(onehot, axis=1)
        idx_ref[s, 0, 0, :] = idx
        qs = z + (zq - z)
        q_acc = qs if q_acc is None else q_acc + qs

    q_ref[0, :, 0, 0, :] = q_acc


def _gmats():
    gp = np.zeros((2, 224, 112), np.float32)
    w = np.arange(112)
    gp[0, 2 * w, w] = 0.5
    gp[0, 2 * w + 1, w] = 0.5
    gp[1, 2 * w, w] = 0.5
    gp[1, 2 * w + 1, w] = -0.5
    return jnp.asarray(gp)


def kernel(x, cb0, cb1, cb2, cb3):
    B, C, H, W = x.shape
    cbs = jnp.stack([cb0, cb1, cb2, cb3])
    xv = x.reshape(B, C, H // 2, 2 * W)
    nblk = B * _HB
    q, idx4, cnt, sq = pl.pallas_call(
        _vrq_body,
        grid=(B, _HB),
        in_specs=[
            pl.BlockSpec((1, C, _TH, 2 * W), lambda b, hb: (b, 0, hb, 0)),
            pl.BlockSpec((4, _K, _C), lambda b, hb: (0, 0, 0)),
            pl.BlockSpec((2, W, W // 2), lambda b, hb: (0, 0, 0)),
        ],
        out_specs=[
            pl.BlockSpec((1, C, 1, 1, _P), lambda b, hb: (b, 0, hb, 0, 0)),
            pl.BlockSpec((4, 1, 1, _P), lambda b, hb: (0, b * _HB + hb, 0, 0)),
            pl.BlockSpec((4, _K), lambda b, hb: (0, 0)),
            pl.BlockSpec((4, 128), lambda b, hb: (0, 0)),
        ],
        out_shape=[
            jax.ShapeDtypeStruct((B, C, _HB, 1, _P), jnp.float32),
            jax.ShapeDtypeStruct((4, nblk, 1, _P), jnp.int32),
            jax.ShapeDtypeStruct((4, _K), jnp.float32),
            jax.ShapeDtypeStruct((4, 128), jnp.float32),
        ],
    )(xv, cbs, _gmats())

    q = q.reshape(B, C, H // 2, W // 2)
    n_pos = B * (H // 2) * (W // 2)
    n_el = n_pos * C
    idxs = idx4.reshape(4, n_pos)
    probs = cnt / jnp.float32(n_pos)
    perps = jnp.exp(-jnp.sum(probs * jnp.log(probs + 1e-10), axis=1))
    m1 = jnp.sum(sq, axis=1) / jnp.float32(n_el)
    losses = m1 + 0.25 * m1
    total_loss = ((losses[0] + losses[1]) + losses[2]) + losses[3]
    return (q, total_loss, perps, idxs)


# speedup vs baseline: 2.4285x; 1.1069x over previous
import jax
import jax.numpy as jnp
import numpy as np
from jax.experimental import pallas as pl

_TH = 8
_P = _TH * 112
_K = 1024
_C = 192
_HB = 112 // _TH


def _split_dot(a, b):
    a0 = a.astype(jnp.bfloat16).astype(jnp.float32)
    r1 = a - a0
    a1 = r1.astype(jnp.bfloat16).astype(jnp.float32)
    a2 = r1 - a1
    dn = (((1,), (0,)), ((), ()))
    out = jax.lax.dot_general(a0, b, dn, preferred_element_type=jnp.float32)
    for part in (a1, a2):
        out = out + jax.lax.dot_general(part, b, dn,
                                        preferred_element_type=jnp.float32)
    return out


def _vrq_body(x_ref, cbs_ref, gp_ref, q_ref, idx_ref, cnt_ref, sq_ref):
    b = pl.program_id(0)
    hb = pl.program_id(1)

    @pl.when(jnp.logical_and(b == 0, hb == 0))
    def _init():
        cnt_ref[...] = jnp.zeros_like(cnt_ref)
        sq_ref[...] = jnp.zeros_like(sq_ref)

    t = x_ref[0]
    top = t[:, :, :224]
    bot = t[:, :, 224:]
    hs = (top + bot).reshape(_C * _TH, 224)
    hd = (top - bot).reshape(_C * _TH, 224)
    g2 = gp_ref[0]
    ps = _split_dot(hs, g2)
    pd = _split_dot(hd, g2)
    subbands = (
        ps[:, :112],
        pd[:, :112],
        ps[:, 112:],
        pd[:, 112:],
    )

    iota_k = jax.lax.broadcasted_iota(jnp.int32, (_K, _P), 0)
    q_acc = None
    for s in range(4):
        z3 = subbands[s].reshape(_C, _TH, 112)
        z = jnp.concatenate([z3[:, th, :] for th in range(_TH)], axis=1)
        cb = cbs_ref[s]
        cn = jnp.sum(cb * cb, axis=1, keepdims=True)
        zn = jnp.sum(z * z, axis=0, keepdims=True)
        m = jax.lax.dot_general(cb, z, (((1,), (0,)), ((), ())),
                                preferred_element_type=jnp.float32)
        dist = (zn + cn) - 2.0 * m
        idx = jnp.argmin(dist, axis=0).astype(jnp.int32)
        onehot = (iota_k == idx[None, :]).astype(jnp.float32)
        zq = jax.lax.dot_general(cb, onehot, (((0,), (0,)), ((), ())),
                                 preferred_element_type=jnp.float32)
        r = zq - z
        sq_ref[s] = sq_ref[s] + jnp.sum((r * r).reshape(_C, _P // 128, 128),
                                        axis=(0, 1))
        cnt_ref[s] = cnt_ref[s] + jnp.sum(onehot, axis=1)
        idx_ref[s, 0, 0, :] = idx
        qs = z + (zq - z)
        q_acc = qs if q_acc is None else q_acc + qs

    q_ref[0, :, 0, 0, :] = q_acc


def _gmats():
    g = np.zeros((1, 224, 224), np.float32)
    w = np.arange(112)
    g[0, 2 * w, w] = 0.5
    g[0, 2 * w + 1, w] = 0.5
    g[0, 2 * w, 112 + w] = 0.5
    g[0, 2 * w + 1, 112 + w] = -0.5
    return jnp.asarray(g)


def kernel(x, cb0, cb1, cb2, cb3):
    B, C, H, W = x.shape
    cbs = jnp.stack([cb0, cb1, cb2, cb3])
    xv = x.reshape(B, C, H // 2, 2 * W)
    nblk = B * _HB
    q, idx4, cnt, sq = pl.pallas_call(
        _vrq_body,
        grid=(B, _HB),
        in_specs=[
            pl.BlockSpec((1, C, _TH, 2 * W), lambda b, hb: (b, 0, hb, 0)),
            pl.BlockSpec((4, _K, _C), lambda b, hb: (0, 0, 0)),
            pl.BlockSpec((1, W, W), lambda b, hb: (0, 0, 0)),
        ],
        out_specs=[
            pl.BlockSpec((1, C, 1, 1, _P), lambda b, hb: (b, 0, hb, 0, 0)),
            pl.BlockSpec((4, 1, 1, _P), lambda b, hb: (0, b * _HB + hb, 0, 0)),
            pl.BlockSpec((4, _K), lambda b, hb: (0, 0)),
            pl.BlockSpec((4, 128), lambda b, hb: (0, 0)),
        ],
        out_shape=[
            jax.ShapeDtypeStruct((B, C, _HB, 1, _P), jnp.float32),
            jax.ShapeDtypeStruct((4, nblk, 1, _P), jnp.int32),
            jax.ShapeDtypeStruct((4, _K), jnp.float32),
            jax.ShapeDtypeStruct((4, 128), jnp.float32),
        ],
    )(xv, cbs, _gmats())

    q = q.reshape(B, C, H // 2, W // 2)
    n_pos = B * (H // 2) * (W // 2)
    n_el = n_pos * C
    idxs = idx4.reshape(4, n_pos)
    probs = cnt / jnp.float32(n_pos)
    perps = jnp.exp(-jnp.sum(probs * jnp.log(probs + 1e-10), axis=1))
    m1 = jnp.sum(sq, axis=1) / jnp.float32(n_el)
    losses = m1 + 0.25 * m1
    total_loss = ((losses[0] + losses[1]) + losses[2]) + losses[3]
    return (q, total_loss, perps, idxs)


# speedup vs baseline: 3.7246x; 1.5337x over previous
import jax
import jax.numpy as jnp
import numpy as np
from jax.experimental import pallas as pl

_TH = 8
_PP = _TH * 128
_K = 1024
_C = 192
_HB = 112 // _TH


def _split_dot(a, b):
    a0 = a.astype(jnp.bfloat16).astype(jnp.float32)
    r1 = a - a0
    a1 = r1.astype(jnp.bfloat16).astype(jnp.float32)
    a2 = r1 - a1
    dn = (((1,), (0,)), ((), ()))
    out = jax.lax.dot_general(a0, b, dn, preferred_element_type=jnp.float32)
    for part in (a1, a2):
        out = out + jax.lax.dot_general(part, b, dn,
                                        preferred_element_type=jnp.float32)
    return out


def _vrq_body(x_ref, cbs_ref, g_ref, q_ref, idx_ref, cnt_ref, sq_ref):
    b = pl.program_id(0)
    hb = pl.program_id(1)

    @pl.when(jnp.logical_and(b == 0, hb == 0))
    def _init():
        cnt_ref[...] = jnp.zeros_like(cnt_ref)
        sq_ref[...] = jnp.zeros_like(sq_ref)

    t = x_ref[0]
    top = t[:, :, :224]
    bot = t[:, :, 224:]
    hs = (top + bot).reshape(_C * _TH, 224)
    hd = (top - bot).reshape(_C * _TH, 224)
    g2 = g_ref[0]
    ps = _split_dot(hs, g2)
    pd = _split_dot(hd, g2)
    subbands = (
        ps[:, :128].reshape(_C, _PP),
        pd[:, :128].reshape(_C, _PP),
        ps[:, 128:].reshape(_C, _PP),
        pd[:, 128:].reshape(_C, _PP),
    )

    iota_k = jax.lax.broadcasted_iota(jnp.int32, (_K, _PP), 0)
    lane = jax.lax.broadcasted_iota(jnp.int32, (1, _PP), 1)
    valid = (lane & 127) < 112
    q_acc = None
    for s in range(4):
        z = subbands[s]
        cb = cbs_ref[s]
        cb2 = cbs_ref[4 + s]
        cn = jnp.sum(cb * cb, axis=1, keepdims=True)
        zn = jnp.sum(z * z, axis=0, keepdims=True)
        m2 = jax.lax.dot_general(cb2, z, (((1,), (0,)), ((), ())),
                                 preferred_element_type=jnp.float32)
        dist = (zn + cn) - m2
        idx = jnp.argmin(dist, axis=0).astype(jnp.int32)
        idx_m = jnp.where(valid[0], idx, -1)
        onehot = (iota_k == idx_m[None, :]).astype(jnp.float32)
        zq = jax.lax.dot_general(cb, onehot, (((0,), (0,)), ((), ())),
                                 preferred_element_type=jnp.float32)
        r = zq - z
        sq_ref[s] = sq_ref[s] + jnp.sum((r * r).reshape(_C, _TH, 128),
                                        axis=(0, 1))
        cnt_ref[:, s:s + 1] = cnt_ref[:, s:s + 1] + jnp.sum(
            onehot, axis=1, keepdims=True)
        idx_ref[s, 0, 0, :] = idx
        qs = z + (zq - z)
        q_acc = qs if q_acc is None else q_acc + qs

    q_ref[0, :, 0, 0, :] = q_acc


def _gmats():
    g = np.zeros((1, 224, 256), np.float32)
    w = np.arange(112)
    g[0, 2 * w, w] = 0.5
    g[0, 2 * w + 1, w] = 0.5
    g[0, 2 * w, 128 + w] = 0.5
    g[0, 2 * w + 1, 128 + w] = -0.5
    return jnp.asarray(g)


def kernel(x, cb0, cb1, cb2, cb3):
    B, C, H, W = x.shape
    cbs = jnp.stack([cb0, cb1, cb2, cb3])
    cbs = jnp.concatenate([cbs, 2.0 * cbs], axis=0)
    xv = x.reshape(B, C, H // 2, 2 * W)
    nblk = B * _HB
    q, idx4, cnt, sq = pl.pallas_call(
        _vrq_body,
        grid=(B, _HB),
        in_specs=[
            pl.BlockSpec((1, C, _TH, 2 * W), lambda b, hb: (b, 0, hb, 0)),
            pl.BlockSpec((8, _K, _C), lambda b, hb: (0, 0, 0)),
            pl.BlockSpec((1, W, 256), lambda b, hb: (0, 0, 0)),
        ],
        out_specs=[
            pl.BlockSpec((1, C, 1, 1, _PP), lambda b, hb: (b, 0, hb, 0, 0)),
            pl.BlockSpec((4, 1, 1, _PP), lambda b, hb: (0, b * _HB + hb, 0, 0)),
            pl.BlockSpec((_K, 128), lambda b, hb: (0, 0)),
            pl.BlockSpec((4, 128), lambda b, hb: (0, 0)),
        ],
        out_shape=[
            jax.ShapeDtypeStruct((B, C, _HB, 1, _PP), jnp.float32),
            jax.ShapeDtypeStruct((4, nblk, 1, _PP), jnp.int32),
            jax.ShapeDtypeStruct((_K, 128), jnp.float32),
            jax.ShapeDtypeStruct((4, 128), jnp.float32),
        ],
    )(xv, cbs, _gmats())

    n_pos = B * (H // 2) * (W // 2)
    n_el = n_pos * C
    q = q.reshape(B, C, _HB, _TH, 128)[..., :112].reshape(B, C, H // 2, W // 2)
    idxs = idx4.reshape(4, nblk, _TH, 128)[..., :112].reshape(4, n_pos)
    counts = cnt[:, :4].T
    probs = counts / jnp.float32(n_pos)
    perps = jnp.exp(-jnp.sum(probs * jnp.log(probs + 1e-10), axis=1))
    m1 = jnp.sum(sq, axis=1) / jnp.float32(n_el)
    losses = m1 + 0.25 * m1
    total_loss = ((losses[0] + losses[1]) + losses[2]) + losses[3]
    return (q, total_loss, perps, idxs)


# speedup vs baseline: 3.7569x; 1.0087x over previous
import jax
import jax.numpy as jnp
import numpy as np
from jax.experimental import pallas as pl
from jax.experimental.pallas import tpu as pltpu

_TH = 8
_PP = _TH * 128
_K = 1024
_C = 192
_HB = 112 // _TH


def _split_dot(a, b):
    a0 = a.astype(jnp.bfloat16).astype(jnp.float32)
    r1 = a - a0
    a1 = r1.astype(jnp.bfloat16).astype(jnp.float32)
    a2 = r1 - a1
    dn = (((1,), (0,)), ((), ()))
    out = jax.lax.dot_general(a0, b, dn, preferred_element_type=jnp.float32)
    for part in (a1, a2):
        out = out + jax.lax.dot_general(part, b, dn,
                                        preferred_element_type=jnp.float32)
    return out


def _vrq_body(x_ref, cbs_ref, g_ref, q_ref, idx_ref, cnt_ref, sq_ref, cn_ref):
    b = pl.program_id(0)
    hb = pl.program_id(1)

    @pl.when(jnp.logical_and(b == 0, hb == 0))
    def _init():
        cnt_ref[...] = jnp.zeros_like(cnt_ref)
        sq_ref[...] = jnp.zeros_like(sq_ref)
        for s in range(4):
            cbi = cbs_ref[s]
            cn_ref[s] = jnp.sum(cbi * cbi, axis=1, keepdims=True)

    t = x_ref[0]
    top = t[:, :, :224]
    bot = t[:, :, 224:]
    hs = (top + bot).reshape(_C * _TH, 224)
    hd = (top - bot).reshape(_C * _TH, 224)
    g2 = g_ref[0]
    ps = _split_dot(hs, g2)
    pd = _split_dot(hd, g2)
    subbands = (
        ps[:, :128].reshape(_C, _PP),
        pd[:, :128].reshape(_C, _PP),
        ps[:, 128:].reshape(_C, _PP),
        pd[:, 128:].reshape(_C, _PP),
    )

    iota_k = jax.lax.broadcasted_iota(jnp.int32, (_K, _PP), 0)
    lane = jax.lax.broadcasted_iota(jnp.int32, (1, _PP), 1)
    valid = (lane & 127) < 112
    q_acc = None
    for s in range(4):
        z = subbands[s]
        cb = cbs_ref[s]
        cb2 = cbs_ref[4 + s]
        cn = cn_ref[s]
        zn = jnp.sum(z * z, axis=0, keepdims=True)
        m2 = jax.lax.dot_general(cb2, z, (((1,), (0,)), ((), ())),
                                 preferred_element_type=jnp.float32)
        dist = (zn + cn) - m2
        idx = jnp.argmin(dist, axis=0).astype(jnp.int32)
        idx_m = jnp.where(valid[0], idx, -1)
        onehot = (iota_k == idx_m[None, :]).astype(jnp.float32)
        zq = jax.lax.dot_general(cb, onehot, (((0,), (0,)), ((), ())),
                                 preferred_element_type=jnp.float32)
        r = zq - z
        sq_ref[s] = sq_ref[s] + jnp.sum((r * r).reshape(_C, _TH, 128),
                                        axis=(0, 1))
        cnt_ref[:, s:s + 1] = cnt_ref[:, s:s + 1] + jnp.sum(
            onehot, axis=1, keepdims=True)
        idx_ref[s, 0, 0, :] = idx
        qs = z + (zq - z)
        q_acc = qs if q_acc is None else q_acc + qs

    q_ref[0, :, 0, 0, :] = q_acc


def _gmats():
    g = np.zeros((1, 224, 256), np.float32)
    w = np.arange(112)
    g[0, 2 * w, w] = 0.5
    g[0, 2 * w + 1, w] = 0.5
    g[0, 2 * w, 128 + w] = 0.5
    g[0, 2 * w + 1, 128 + w] = -0.5
    return jnp.asarray(g)


def kernel(x, cb0, cb1, cb2, cb3):
    B, C, H, W = x.shape
    cbs = jnp.stack([cb0, cb1, cb2, cb3])
    cbs = jnp.concatenate([cbs, 2.0 * cbs], axis=0)
    xv = x.reshape(B, C, H // 2, 2 * W)
    nblk = B * _HB
    q, idx4, cnt, sq = pl.pallas_call(
        _vrq_body,
        grid=(B, _HB),
        in_specs=[
            pl.BlockSpec((1, C, _TH, 2 * W), lambda b, hb: (b, 0, hb, 0)),
            pl.BlockSpec((8, _K, _C), lambda b, hb: (0, 0, 0)),
            pl.BlockSpec((1, W, 256), lambda b, hb: (0, 0, 0)),
        ],
        out_specs=[
            pl.BlockSpec((1, C, 1, 1, _PP), lambda b, hb: (b, 0, hb, 0, 0)),
            pl.BlockSpec((4, 1, 1, _PP), lambda b, hb: (0, b * _HB + hb, 0, 0)),
            pl.BlockSpec((_K, 128), lambda b, hb: (0, 0)),
            pl.BlockSpec((4, 128), lambda b, hb: (0, 0)),
        ],
        out_shape=[
            jax.ShapeDtypeStruct((B, C, _HB, 1, _PP), jnp.float32),
            jax.ShapeDtypeStruct((4, nblk, 1, _PP), jnp.int32),
            jax.ShapeDtypeStruct((_K, 128), jnp.float32),
            jax.ShapeDtypeStruct((4, 128), jnp.float32),
        ],
        scratch_shapes=[pltpu.VMEM((4, _K, 1), jnp.float32)],
    )(xv, cbs, _gmats())

    n_pos = B * (H // 2) * (W // 2)
    n_el = n_pos * C
    q = q.reshape(B, C, _HB, _TH, 128)[..., :112].reshape(B, C, H // 2, W // 2)
    idxs = idx4.reshape(4, nblk, _TH, 128)[..., :112].reshape(4, n_pos)
    counts = cnt[:, :4].T
    probs = counts / jnp.float32(n_pos)
    perps = jnp.exp(-jnp.sum(probs * jnp.log(probs + 1e-10), axis=1))
    m1 = jnp.sum(sq, axis=1) / jnp.float32(n_el)
    losses = m1 + 0.25 * m1
    total_loss = ((losses[0] + losses[1]) + losses[2]) + losses[3]
    return (q, total_loss, perps, idxs)
